# Initial kernel scaffold; baseline (speedup 1.0000x reference)
import jax
import jax.numpy as jnp
from jax.experimental import pallas as pl


def kernel(vid_feat, txt_feat, W_vid, b_vid, W_txt, b_txt, W_stage1, b_stage1, Wa, ba, Wb, bb, W_prop, b_prop, W_score, b_score, txt_mask):
    raise NotImplementedError("write your pallas kernel here")


# speedup vs baseline: 1.3510x; 1.3510x over previous
import functools

import jax
import jax.numpy as jnp
from jax import lax
from jax.experimental import pallas as pl
from jax.experimental.pallas import tpu as pltpu
from jax.experimental.pallas import tpu_sc as plsc

_B, _T, _D, _LT, _NQ = 32, 1024, 1024, 32, 32
_BT = 512
_HI = lax.Precision.HIGHEST


def _dot_std(x, w):
    return jnp.dot(x.astype(jnp.bfloat16), w.astype(jnp.bfloat16),
                   preferred_element_type=jnp.float32)


def _txt_pool_body(txt_ref, wt_ref, bt_ref, tm_ref, out_ref):
    y = jnp.tanh(_dot_std(txt_ref[...], wt_ref[...]) + bt_ref[...]
                 ) * tm_ref[...]
    r_iota = lax.broadcasted_iota(jnp.int32, (_B, _B * _LT), 1)
    b_iota = lax.broadcasted_iota(jnp.int32, (_B, _B * _LT), 0)
    sel = (r_iota // _LT == b_iota).astype(jnp.float32)
    ssum = jnp.dot(sel, y, precision=_HI, preferred_element_type=jnp.float32)
    cnt = jnp.dot(sel, tm_ref[...], precision=_HI,
                  preferred_element_type=jnp.float32)
    out_ref[...] = ssum / jnp.maximum(cnt, 1.0)


def _logits_body(x_ref, tp_ref, wv_ref, bv_ref, w1_ref, b1_ref, wa_ref,
                 ba_ref, wb_ref, bb_ref, out_ref):
    x = x_ref[...]
    v = jax.nn.relu(_dot_std(x, wv_ref[...]) + bv_ref[...]) + tp_ref[0]
    h = _dot_std(v, w1_ref[...]) + b1_ref[...]
    g = jax.nn.relu(_dot_std(h, wa_ref[...]) + ba_ref[...])
    out_ref[...] = _dot_std(g, wb_ref[...]) + bb_ref[...]


def _topk_body(lg_ref, ctr_ref, gidx_ref):
    vals = lg_ref[...]
    iota = lax.broadcasted_iota(jnp.int32, (_B, _T), 1)
    b_off = lax.broadcasted_iota(jnp.int32, (_B, 1), 0) * _T
    for k in range(_NQ):
        m = jnp.max(vals, axis=1, keepdims=True)
        amin = jnp.min(jnp.where(vals == m, iota, _T), axis=1, keepdims=True)
        ctr = amin.astype(jnp.float32) / float(_T)
        gi = jnp.clip(jnp.round(ctr * float(_T - 1)).astype(jnp.int32),
                      0, _T - 1)
        ctr_ref[:, k:k + 1] = ctr
        gidx_ref[:, k:k + 1] = gi + b_off
        vals = jnp.where(iota == amin, -jnp.inf, vals)


def _head_body(fr_ref, tp_ref, wv_ref, bv_ref, wp_ref, bp_ref, ws_ref,
               bs_ref, ctr_ref, out_ref):
    r_iota = lax.broadcasted_iota(jnp.int32, (_B * _NQ, _B), 0)
    b_iota = lax.broadcasted_iota(jnp.int32, (_B * _NQ, _B), 1)
    selT = (r_iota // _NQ == b_iota).astype(jnp.float32)
    tp_rep = jnp.dot(selT, tp_ref[...], precision=_HI,
                     preferred_element_type=jnp.float32)
    fq = jax.nn.relu(_dot_std(fr_ref[...], wv_ref[...]) + bv_ref[...]
                     ) + tp_rep
    offs = jnp.tanh(_dot_std(fq, wp_ref[...]) + bp_ref[...]) * 0.5
    sc = _dot_std(fq, ws_ref[...]) + bs_ref[...]
    c = ctr_ref[...]
    st = jnp.clip(c - 0.05 + offs[:, 0:1], 0.0, 1.0)
    ed = jnp.clip(c + 0.05 + offs[:, 1:2], 0.0, 1.0)
    out_ref[...] = jnp.concatenate([st, ed, sc], axis=1)


def _sc_gather(table, flat_idx):
    info = plsc.get_sparse_core_info()
    nc, ns = info.num_cores, info.num_subcores
    nw = nc * ns
    n, d = flat_idx.shape[0], table.shape[1]
    b_per_w = n // nw
    mesh = plsc.VectorSubcoreMesh(core_axis_name="c", subcore_axis_name="s")

    @functools.partial(
        pl.kernel, mesh=mesh,
        out_type=jax.ShapeDtypeStruct((n, d), jnp.float32),
        scratch_types=[
            pltpu.VMEM((b_per_w,), jnp.int32),
            pltpu.VMEM((b_per_w, d), jnp.float32),
            pltpu.SemaphoreType.DMA,
        ],
    )
    def gather_k(table_hbm, idx_hbm, out_hbm, idx_v, rows_v, sem):
        wid = lax.axis_index("s") * nc + lax.axis_index("c")
        base = wid * b_per_w
        pltpu.sync_copy(idx_hbm.at[pl.ds(base, b_per_w)], idx_v)
        pltpu.async_copy(table_hbm.at[idx_v], rows_v, sem).wait()
        pltpu.sync_copy(rows_v, out_hbm.at[pl.ds(base, b_per_w)])

    return gather_k(table, flat_idx)


def kernel(vid_feat, txt_feat, W_vid, b_vid, W_txt, b_txt, W_stage1,
           b_stage1, Wa, ba, Wb, bb, W_prop, b_prop, W_score, b_score,
           txt_mask):
    B, T, D = vid_feat.shape
    LT = txt_feat.shape[1]
    NQ = _NQ

    tmf = txt_mask.astype(jnp.float32)
    txt = jnp.tanh(txt_feat @ W_txt + b_txt) * tmf[..., None]
    txt_pool = txt.sum(axis=1) / jnp.maximum(tmf.sum(axis=1)[:, None], 1.0)

    vid2 = vid_feat.reshape(B * T, D)
    nblk = (B * T) // _BT
    logits = pl.pallas_call(
        _logits_body,
        grid=(nblk,),
        in_specs=[
            pl.BlockSpec((_BT, D), lambda i: (i, 0)),
            pl.BlockSpec((1, 1, D), lambda i: (i * _BT // T, 0, 0)),
            pl.BlockSpec((D, D), lambda i: (0, 0)),
            pl.BlockSpec((1, D), lambda i: (0, 0)),
            pl.BlockSpec((D, D), lambda i: (0, 0)),
            pl.BlockSpec((1, D), lambda i: (0, 0)),
            pl.BlockSpec((D, D), lambda i: (0, 0)),
            pl.BlockSpec((1, D), lambda i: (0, 0)),
            pl.BlockSpec((D, 1), lambda i: (0, 0)),
            pl.BlockSpec((1, 1), lambda i: (0, 0)),
        ],
        out_specs=pl.BlockSpec((_BT, 1), lambda i: (i, 0)),
        out_shape=jax.ShapeDtypeStruct((B * T, 1), jnp.float32),
        compiler_params=pltpu.CompilerParams(
            dimension_semantics=("arbitrary",)),
    )(vid2, txt_pool.reshape(B, 1, D), W_vid, b_vid.reshape(1, D),
      W_stage1[:, 2 * D:], b_stage1[2 * D:].reshape(1, D), Wa[2],
      ba[2].reshape(1, D), Wb[2], bb[2].reshape(1, 1))

    ctr, gidx = pl.pallas_call(
        _topk_body,
        out_shape=[
            jax.ShapeDtypeStruct((B, NQ), jnp.float32),
            jax.ShapeDtypeStruct((B, NQ), jnp.int32),
        ],
    )(logits.reshape(B, T))

    feat_raw = _sc_gather(vid2, gidx.reshape(B * NQ))

    out = pl.pallas_call(
        _head_body,
        out_shape=jax.ShapeDtypeStruct((B * NQ, 3), jnp.float32),
    )(feat_raw, txt_pool, W_vid, b_vid.reshape(1, D), W_prop[2],
      b_prop[2].reshape(1, 2), W_score[2], b_score[2].reshape(1, 1),
      ctr.reshape(B * NQ, 1))

    return out.reshape(B, NQ, 3)
